# Initial kernel scaffold; baseline (speedup 1.0000x reference)
import jax
import jax.numpy as jnp
from jax.experimental import pallas as pl


def kernel(x, edge_index, Wl0, bl0, Wr0, Wl1, bl1, Wr1, Wl2, bl2, Wr2):
    raise NotImplementedError("write your pallas kernel here")


# speedup vs baseline: 5.0779x; 5.0779x over previous
import functools

import jax
import jax.numpy as jnp
from jax import lax
from jax.experimental import pallas as pl
from jax.experimental.pallas import tpu as pltpu
from jax.experimental.pallas import tpu_sc as plsc

_NC = 2
_NS = 16
_L = 16


def _make_agg(N, D, E, with_deg, NP):
    NW = _NC * _NS
    EPW = E // NW
    C = 80
    NCH = EPW // C
    RPT = NP // _NS
    ZR = 128
    DPT = NP // _NS
    mesh = plsc.VectorSubcoreMesh(core_axis_name="c", subcore_axis_name="s")

    out_type = [jax.ShapeDtypeStruct((_NC, NP, D), jnp.float32)]
    scratch = [
        pltpu.VMEM_SHARED((NP, D), jnp.float32),
        pltpu.VMEM((ZR, D), jnp.float32),
        pltpu.VMEM((C,), jnp.int32),
        pltpu.VMEM((C,), jnp.int32),
        pltpu.VMEM((C, D), jnp.float32),
        pltpu.SemaphoreType.DMA,
    ]
    if with_deg:
        out_type.append(jax.ShapeDtypeStruct((_NC * NP,), jnp.float32))
        scratch += [
            pltpu.VMEM_SHARED((NP,), jnp.float32),
            pltpu.VMEM((C,), jnp.float32),
            pltpu.VMEM((DPT,), jnp.float32),
        ]

    @functools.partial(pl.kernel, out_type=out_type, mesh=mesh,
                       scratch_types=scratch)
    def agg(*refs):
        if with_deg:
            (y_hbm, src_hbm, dst_hbm, out_hbm, deg_hbm,
             acc, zbuf, srcv, dstv, rows, sem, dacc, ones, dzero) = refs
        else:
            (y_hbm, src_hbm, dst_hbm, out_hbm,
             acc, zbuf, srcv, dstv, rows, sem) = refs
        c = lax.axis_index("c")
        s = lax.axis_index("s")
        wid = s * _NC + c
        zv = jnp.zeros((_L,), jnp.float32)

        def zrow(i, carry):
            for j in range(D // _L):
                zbuf[i, pl.ds(j * _L, _L)] = zv
            return carry
        lax.fori_loop(0, ZR, zrow, 0)
        for k in range(RPT // ZR):
            pltpu.sync_copy(zbuf, acc.at[pl.ds(s * RPT + k * ZR, ZR)])
        if with_deg:
            ov = jnp.full((_L,), 1.0, jnp.float32)
            for i in range(C // _L):
                ones[pl.ds(i * _L, _L)] = ov
            def dzrow(i, carry):
                dzero[pl.ds(i * _L, _L)] = zv
                return carry
            lax.fori_loop(0, DPT // _L, dzrow, 0)
            pltpu.sync_copy(dzero, dacc.at[pl.ds(s * DPT, DPT)])
        plsc.subcore_barrier()

        ebase = wid * EPW

        def chunk(g, carry):
            off = pl.multiple_of(ebase + g * C, 8)
            pltpu.sync_copy(src_hbm.at[pl.ds(off, C)], srcv)
            pltpu.sync_copy(dst_hbm.at[pl.ds(off, C)], dstv)
            pltpu.async_copy(y_hbm.at[srcv], rows, sem).wait()
            pltpu.sync_copy(rows, acc.at[dstv], add=True)
            if with_deg:
                pltpu.sync_copy(ones, dacc.at[dstv], add=True)
            return carry
        lax.fori_loop(0, NCH, chunk, 0)

        plsc.subcore_barrier()
        pltpu.sync_copy(acc.at[pl.ds(s * RPT, RPT)],
                        out_hbm.at[c, pl.ds(s * RPT, RPT)])
        if with_deg:
            pltpu.sync_copy(dacc.at[pl.ds(s * DPT, DPT)], dzero)
            pltpu.sync_copy(dzero, deg_hbm.at[pl.ds(c * NP + s * DPT, DPT)])

    return agg


def _proj(x, Wl, Wr, bl):
    N, Din = x.shape
    Do = Wl.shape[1]
    BN = 1000

    def body(x_ref, wl_ref, wr_ref, b_ref, y_ref, z_ref):
        xb = x_ref[...]
        y_ref[...] = jnp.dot(xb, wl_ref[...],
                             preferred_element_type=jnp.float32)
        z_ref[...] = jnp.dot(xb, wr_ref[...],
                             preferred_element_type=jnp.float32) + b_ref[...]

    y, z = pl.pallas_call(
        body,
        grid=(N // BN,),
        in_specs=[
            pl.BlockSpec((BN, Din), lambda i: (i, 0)),
            pl.BlockSpec((Din, Do), lambda i: (0, 0)),
            pl.BlockSpec((Din, Do), lambda i: (0, 0)),
            pl.BlockSpec((1, Do), lambda i: (0, 0)),
        ],
        out_specs=[
            pl.BlockSpec((BN, Do), lambda i: (i, 0)),
            pl.BlockSpec((BN, Do), lambda i: (i, 0)),
        ],
        out_shape=[jax.ShapeDtypeStruct((N, Do), jnp.float32)] * 2,
    )(x, Wl, Wr, bl.reshape(1, -1))
    return y, z


def _combine(p0, p1, z, d0, d1, act):
    N, Do = z.shape
    BN = 1000

    def body(p0_ref, p1_ref, z_ref, d0_ref, d1_ref, o_ref):
        deg = jnp.maximum(d0_ref[...] + d1_ref[...], 1.0)
        u = (p0_ref[...] + p1_ref[...]) / deg + z_ref[...]
        if act == "relu":
            o_ref[...] = jnp.maximum(u, 0.0)
        else:
            m = jnp.max(u, axis=1, keepdims=True)
            e = u - m
            o_ref[...] = e - jnp.log(
                jnp.sum(jnp.exp(e), axis=1, keepdims=True))

    return pl.pallas_call(
        body,
        grid=(N // BN,),
        in_specs=[
            pl.BlockSpec((BN, Do), lambda i: (i, 0)),
            pl.BlockSpec((BN, Do), lambda i: (i, 0)),
            pl.BlockSpec((BN, Do), lambda i: (i, 0)),
            pl.BlockSpec((BN, 1), lambda i: (i, 0)),
            pl.BlockSpec((BN, 1), lambda i: (i, 0)),
        ],
        out_specs=pl.BlockSpec((BN, Do), lambda i: (i, 0)),
        out_shape=jax.ShapeDtypeStruct((N, Do), jnp.float32),
    )(p0, p1, z, d0, d1)


def _final(h, p0, p1, d0, d1, Wl, Wr, bl):
    N, Dh = h.shape
    Do = Wl.shape[1]
    BN = 1000

    def body(h_ref, p0_ref, p1_ref, d0_ref, d1_ref, wl_ref, wr_ref, b_ref,
             o_ref):
        deg = jnp.maximum(d0_ref[...] + d1_ref[...], 1.0)
        m = (p0_ref[...] + p1_ref[...]) / deg
        u = (jnp.dot(m, wl_ref[...], preferred_element_type=jnp.float32)
             + jnp.dot(h_ref[...], wr_ref[...],
                       preferred_element_type=jnp.float32) + b_ref[...])
        mx = jnp.max(u, axis=1, keepdims=True)
        e = u - mx
        o_ref[...] = e - jnp.log(jnp.sum(jnp.exp(e), axis=1, keepdims=True))

    return pl.pallas_call(
        body,
        grid=(N // BN,),
        in_specs=[
            pl.BlockSpec((BN, Dh), lambda i: (i, 0)),
            pl.BlockSpec((BN, Dh), lambda i: (i, 0)),
            pl.BlockSpec((BN, Dh), lambda i: (i, 0)),
            pl.BlockSpec((BN, 1), lambda i: (i, 0)),
            pl.BlockSpec((BN, 1), lambda i: (i, 0)),
            pl.BlockSpec((Dh, Do), lambda i: (0, 0)),
            pl.BlockSpec((Dh, Do), lambda i: (0, 0)),
            pl.BlockSpec((1, Do), lambda i: (0, 0)),
        ],
        out_specs=pl.BlockSpec((BN, Do), lambda i: (i, 0)),
        out_shape=jax.ShapeDtypeStruct((N, Do), jnp.float32),
    )(h, p0, p1, d0, d1, Wl, Wr, bl.reshape(1, -1))


def kernel(x, edge_index, Wl0, bl0, Wr0, Wl1, bl1, Wr1, Wl2, bl2, Wr2):
    N, Din = x.shape
    E = edge_index.shape[1]
    Dh = Wl0.shape[1]
    NP = ((N + 128 * _NS - 1) // (128 * _NS)) * (128 * _NS)
    src = edge_index[0]
    dst = edge_index[1]

    agg_deg = _make_agg(N, Dh, E, True, NP)
    agg_h = _make_agg(N, Dh, E, False, NP)

    y, z = _proj(x, Wl0, Wr0, bl0)
    p, degf = agg_deg(y, src, dst)
    degp = degf.reshape(_NC, NP)
    d0 = degp[0, :N].reshape(N, 1)
    d1 = degp[1, :N].reshape(N, 1)
    h = _combine(p[0, :N], p[1, :N], z, d0, d1, "relu")

    y, z = _proj(h, Wl1, Wr1, bl1)
    (p,) = agg_h(y, src, dst)
    h = _combine(p[0, :N], p[1, :N], z, d0, d1, "relu")

    (p,) = agg_h(h, src, dst)
    return _final(h, p[0, :N], p[1, :N], d0, d1, Wl2, Wr2, bl2)


# speedup vs baseline: 11.1579x; 2.1974x over previous
import functools

import jax
import jax.numpy as jnp
from jax import lax
from jax.experimental import pallas as pl
from jax.experimental.pallas import tpu as pltpu
from jax.experimental.pallas import tpu_sc as plsc

_NC = 2
_NS = 16
_L = 16


def _make_agg(N, D, E, with_deg, NP):
    NW = _NC * _NS
    EPW = E // NW
    C = 80
    NCH = EPW // C
    K = 2
    FULL = NP // _NS
    LAST = N - FULL * (_NS - 1)
    mesh = plsc.VectorSubcoreMesh(core_axis_name="c", subcore_axis_name="s")

    out_type = [jax.ShapeDtypeStruct((_NC, N, D), jnp.float32)]
    scratch = [
        pltpu.VMEM_SHARED((N, D), jnp.float32),
        pltpu.VMEM((EPW,), jnp.int32),
        pltpu.VMEM((NCH, C), jnp.int32),
        [pltpu.VMEM((C, D), jnp.float32) for _ in range(K)],
        [pltpu.SemaphoreType.DMA for _ in range(K)],
    ]
    if with_deg:
        out_type.append(jax.ShapeDtypeStruct((_NC * N,), jnp.float32))
        scratch += [
            pltpu.VMEM_SHARED((N,), jnp.float32),
            pltpu.VMEM((C,), jnp.float32),
            pltpu.VMEM((FULL,), jnp.float32),
        ]

    @functools.partial(pl.kernel, out_type=out_type, mesh=mesh,
                       scratch_types=scratch)
    def agg(*refs):
        if with_deg:
            (y_hbm, src_hbm, dst_hbm, out_hbm, deg_hbm,
             acc, srcv, dstv, rows, sems, dacc, ones, dzero) = refs
        else:
            (y_hbm, src_hbm, dst_hbm, out_hbm,
             acc, srcv, dstv, rows, sems) = refs
        c = lax.axis_index("c")
        s = lax.axis_index("s")
        wid = s * _NC + c
        zv = jnp.zeros((_L,), jnp.float32)

        off = pl.multiple_of(wid * EPW, 8)
        ih0 = pltpu.async_copy(src_hbm.at[pl.ds(off, EPW)], srcv, sems[0])
        ih1 = pltpu.async_copy(dst_hbm.at[wid], dstv, sems[1])

        def zrow(i, carry):
            for b in range(K):
                for j in range(D // _L):
                    rows[b][i, pl.ds(j * _L, _L)] = zv
            return carry
        lax.fori_loop(0, C, zrow, 0)
        nz_full = FULL // C
        nz_last = LAST // C
        for k in range(nz_full):
            zcopy = lambda: pltpu.sync_copy(
                rows[k % K], acc.at[pl.ds(s * FULL + k * C, C)])
            if k < nz_last:
                zcopy()
            else:
                pl.when(s < _NS - 1)(zcopy)
        if with_deg:
            ov = jnp.full((_L,), 1.0, jnp.float32)
            for i in range(C // _L):
                ones[pl.ds(i * _L, _L)] = ov
            def dzrow(i, carry):
                dzero[pl.ds(i * _L, _L)] = zv
                return carry
            lax.fori_loop(0, FULL // _L, dzrow, 0)
            @pl.when(s < _NS - 1)
            def _():
                pltpu.sync_copy(dzero, dacc.at[pl.ds(s * FULL, FULL)])
            @pl.when(s == _NS - 1)
            def _():
                pltpu.sync_copy(dzero.at[pl.ds(0, LAST)],
                                dacc.at[pl.ds(s * FULL, LAST)])
        ih0.wait()
        ih1.wait()
        plsc.subcore_barrier()

        def fire(g, b):
            pltpu.async_copy(
                y_hbm.at[srcv.at[pl.ds(pl.multiple_of(g * C, 8), C)]],
                rows[b], sems[b])

        def drain(g, b):
            pltpu.make_async_copy(
                y_hbm.at[srcv.at[pl.ds(pl.multiple_of(g * C, 8), C)]],
                rows[b], sems[b]).wait()
            pltpu.sync_copy(rows[b], acc.at[dstv.at[g]], add=True)
            if with_deg:
                pltpu.sync_copy(ones, dacc.at[dstv.at[g]], add=True)

        fire(0, 0)
        fire(1, 1)

        def grp(gg, carry):
            g0 = gg * 2
            drain(g0, 0)
            @pl.when(g0 + 2 < NCH)
            def _():
                fire(g0 + 2, 0)
            drain(g0 + 1, 1)
            @pl.when(g0 + 3 < NCH)
            def _():
                fire(g0 + 3, 1)
            return carry
        lax.fori_loop(0, NCH // 2, grp, 0)
        if NCH % 2:
            drain(NCH - 1, 0)

        plsc.subcore_barrier()
        @pl.when(s < _NS - 1)
        def _():
            pltpu.sync_copy(acc.at[pl.ds(s * FULL, FULL)],
                            out_hbm.at[c, pl.ds(s * FULL, FULL)])
        @pl.when(s == _NS - 1)
        def _():
            pltpu.sync_copy(acc.at[pl.ds(s * FULL, LAST)],
                            out_hbm.at[c, pl.ds(s * FULL, LAST)])
        if with_deg:
            @pl.when(s < _NS - 1)
            def _():
                pltpu.sync_copy(dacc.at[pl.ds(s * FULL, FULL)], dzero)
                pltpu.sync_copy(
                    dzero, deg_hbm.at[pl.ds(c * N + s * FULL, FULL)])
            @pl.when(s == _NS - 1)
            def _():
                pltpu.sync_copy(dacc.at[pl.ds(s * FULL, LAST)],
                                dzero.at[pl.ds(0, LAST)])
                pltpu.sync_copy(
                    dzero.at[pl.ds(0, LAST)],
                    deg_hbm.at[pl.ds(c * N + s * FULL, LAST)])

    return agg


def _proj(x, Wl, Wr, bl):
    N, Din = x.shape
    Do = Wl.shape[1]
    BN = 1000

    def body(x_ref, wl_ref, wr_ref, b_ref, y_ref, z_ref):
        xb = x_ref[...]
        y_ref[...] = jnp.dot(xb, wl_ref[...],
                             preferred_element_type=jnp.float32)
        z_ref[...] = jnp.dot(xb, wr_ref[...],
                             preferred_element_type=jnp.float32) + b_ref[...]

    y, z = pl.pallas_call(
        body,
        grid=(N // BN,),
        in_specs=[
            pl.BlockSpec((BN, Din), lambda i: (i, 0)),
            pl.BlockSpec((Din, Do), lambda i: (0, 0)),
            pl.BlockSpec((Din, Do), lambda i: (0, 0)),
            pl.BlockSpec((1, Do), lambda i: (0, 0)),
        ],
        out_specs=[
            pl.BlockSpec((BN, Do), lambda i: (i, 0)),
            pl.BlockSpec((BN, Do), lambda i: (i, 0)),
        ],
        out_shape=[jax.ShapeDtypeStruct((N, Do), jnp.float32)] * 2,
    )(x, Wl, Wr, bl.reshape(1, -1))
    return y, z


def _combine(p0, p1, z, d0, d1, act):
    N, Do = z.shape
    BN = 1000

    def body(p0_ref, p1_ref, z_ref, d0_ref, d1_ref, o_ref):
        deg = jnp.maximum(d0_ref[...] + d1_ref[...], 1.0)
        u = (p0_ref[...] + p1_ref[...]) / deg + z_ref[...]
        if act == "relu":
            o_ref[...] = jnp.maximum(u, 0.0)
        else:
            m = jnp.max(u, axis=1, keepdims=True)
            e = u - m
            o_ref[...] = e - jnp.log(
                jnp.sum(jnp.exp(e), axis=1, keepdims=True))

    return pl.pallas_call(
        body,
        grid=(N // BN,),
        in_specs=[
            pl.BlockSpec((BN, Do), lambda i: (i, 0)),
            pl.BlockSpec((BN, Do), lambda i: (i, 0)),
            pl.BlockSpec((BN, Do), lambda i: (i, 0)),
            pl.BlockSpec((BN, 1), lambda i: (i, 0)),
            pl.BlockSpec((BN, 1), lambda i: (i, 0)),
        ],
        out_specs=pl.BlockSpec((BN, Do), lambda i: (i, 0)),
        out_shape=jax.ShapeDtypeStruct((N, Do), jnp.float32),
    )(p0, p1, z, d0, d1)


def _final(h, p0, p1, d0, d1, Wl, Wr, bl):
    N, Dh = h.shape
    Do = Wl.shape[1]
    BN = 1000

    def body(h_ref, p0_ref, p1_ref, d0_ref, d1_ref, wl_ref, wr_ref, b_ref,
             o_ref):
        deg = jnp.maximum(d0_ref[...] + d1_ref[...], 1.0)
        m = (p0_ref[...] + p1_ref[...]) / deg
        u = (jnp.dot(m, wl_ref[...], preferred_element_type=jnp.float32)
             + jnp.dot(h_ref[...], wr_ref[...],
                       preferred_element_type=jnp.float32) + b_ref[...])
        mx = jnp.max(u, axis=1, keepdims=True)
        e = u - mx
        o_ref[...] = e - jnp.log(jnp.sum(jnp.exp(e), axis=1, keepdims=True))

    return pl.pallas_call(
        body,
        grid=(N // BN,),
        in_specs=[
            pl.BlockSpec((BN, Dh), lambda i: (i, 0)),
            pl.BlockSpec((BN, Dh), lambda i: (i, 0)),
            pl.BlockSpec((BN, Dh), lambda i: (i, 0)),
            pl.BlockSpec((BN, 1), lambda i: (i, 0)),
            pl.BlockSpec((BN, 1), lambda i: (i, 0)),
            pl.BlockSpec((Dh, Do), lambda i: (0, 0)),
            pl.BlockSpec((Dh, Do), lambda i: (0, 0)),
            pl.BlockSpec((1, Do), lambda i: (0, 0)),
        ],
        out_specs=pl.BlockSpec((BN, Do), lambda i: (i, 0)),
        out_shape=jax.ShapeDtypeStruct((N, Do), jnp.float32),
    )(h, p0, p1, d0, d1, Wl, Wr, bl.reshape(1, -1))


def kernel(x, edge_index, Wl0, bl0, Wr0, Wl1, bl1, Wr1, Wl2, bl2, Wr2):
    N, Din = x.shape
    E = edge_index.shape[1]
    Dh = Wl0.shape[1]
    NP = ((N + 128 * _NS - 1) // (128 * _NS)) * (128 * _NS)
    NW = _NC * _NS
    C = 80
    src = edge_index[0]
    dst = edge_index[1].reshape(NW, E // (NW * C), C)

    agg_deg = _make_agg(N, Dh, E, True, NP)
    agg_h = _make_agg(N, Dh, E, False, NP)

    y, z = _proj(x, Wl0, Wr0, bl0)
    p, degf = agg_deg(y, src, dst)
    degp = degf.reshape(_NC, N)
    d0 = degp[0].reshape(N, 1)
    d1 = degp[1].reshape(N, 1)
    h = _combine(p[0], p[1], z, d0, d1, "relu")

    y, z = _proj(h, Wl1, Wr1, bl1)
    (p,) = agg_h(y, src, dst)
    h = _combine(p[0], p[1], z, d0, d1, "relu")

    (p,) = agg_h(h, src, dst)
    return _final(h, p[0], p[1], d0, d1, Wl2, Wr2, bl2)


# speedup vs baseline: 11.3099x; 1.0136x over previous
import functools

import jax
import jax.numpy as jnp
from jax import lax
from jax.experimental import pallas as pl
from jax.experimental.pallas import tpu as pltpu
from jax.experimental.pallas import tpu_sc as plsc

_NC = 2
_NS = 16
_L = 16


def _make_agg(N, D, E, with_deg, NP):
    NW = _NC * _NS
    EPW = E // NW
    C = 80
    NCH = EPW // C
    K = 2
    FULL = NP // _NS
    LAST = N - FULL * (_NS - 1)
    mesh = plsc.VectorSubcoreMesh(core_axis_name="c", subcore_axis_name="s")

    out_type = [jax.ShapeDtypeStruct((_NC, N, D), jnp.float32)]
    scratch = [
        pltpu.VMEM_SHARED((N, D), jnp.float32),
        pltpu.VMEM((EPW,), jnp.int32),
        pltpu.VMEM((NCH, C), jnp.int32),
        [pltpu.VMEM((C, D), jnp.float32) for _ in range(K)],
        [pltpu.SemaphoreType.DMA for _ in range(K)],
    ]
    if with_deg:
        out_type.append(jax.ShapeDtypeStruct((_NC * N,), jnp.float32))
        scratch += [
            pltpu.VMEM_SHARED((N,), jnp.float32),
            pltpu.VMEM((C,), jnp.float32),
            pltpu.VMEM((FULL,), jnp.float32),
        ]

    @functools.partial(pl.kernel, out_type=out_type, mesh=mesh,
                       scratch_types=scratch)
    def agg(*refs):
        if with_deg:
            (y_hbm, src_hbm, dst_hbm, out_hbm, deg_hbm,
             acc, srcv, dstv, rows, sems, dacc, ones, dzero) = refs
        else:
            (y_hbm, src_hbm, dst_hbm, out_hbm,
             acc, srcv, dstv, rows, sems) = refs
        c = lax.axis_index("c")
        s = lax.axis_index("s")
        wid = s * _NC + c
        zv = jnp.zeros((_L,), jnp.float32)

        off = pl.multiple_of(wid * EPW, 8)
        ih0 = pltpu.async_copy(src_hbm.at[pl.ds(off, EPW)], srcv, sems[0])
        ih1 = pltpu.async_copy(dst_hbm.at[wid], dstv, sems[1])

        def zrow(i, carry):
            for b in range(K):
                for j in range(D // _L):
                    rows[b][i, pl.ds(j * _L, _L)] = zv
            return carry
        lax.fori_loop(0, C, zrow, 0)
        nz_full = FULL // C
        nz_last = LAST // C
        for k in range(nz_full):
            zcopy = lambda: pltpu.sync_copy(
                rows[k % K], acc.at[pl.ds(s * FULL + k * C, C)])
            if k < nz_last:
                zcopy()
            else:
                pl.when(s < _NS - 1)(zcopy)
        if with_deg:
            ov = jnp.full((_L,), 1.0, jnp.float32)
            for i in range(C // _L):
                ones[pl.ds(i * _L, _L)] = ov
            def dzrow(i, carry):
                dzero[pl.ds(i * _L, _L)] = zv
                return carry
            lax.fori_loop(0, FULL // _L, dzrow, 0)
            @pl.when(s < _NS - 1)
            def _():
                pltpu.sync_copy(dzero, dacc.at[pl.ds(s * FULL, FULL)])
            @pl.when(s == _NS - 1)
            def _():
                pltpu.sync_copy(dzero.at[pl.ds(0, LAST)],
                                dacc.at[pl.ds(s * FULL, LAST)])
        ih0.wait()
        ih1.wait()
        plsc.subcore_barrier()

        def fire(g, b):
            pltpu.async_copy(
                y_hbm.at[srcv.at[pl.ds(pl.multiple_of(g * C, 8), C)]],
                rows[b], sems[b])

        def drain(g, b):
            pltpu.make_async_copy(
                y_hbm.at[srcv.at[pl.ds(pl.multiple_of(g * C, 8), C)]],
                rows[b], sems[b]).wait()
            pltpu.sync_copy(rows[b], acc.at[dstv.at[g]], add=True)
            if with_deg:
                pltpu.sync_copy(ones, dacc.at[dstv.at[g]], add=True)

        fire(0, 0)
        fire(1, 1)

        def grp(gg, carry):
            g0 = gg * 2
            drain(g0, 0)
            @pl.when(g0 + 2 < NCH)
            def _():
                fire(g0 + 2, 0)
            drain(g0 + 1, 1)
            @pl.when(g0 + 3 < NCH)
            def _():
                fire(g0 + 3, 1)
            return carry
        lax.fori_loop(0, NCH // 2, grp, 0)
        if NCH % 2:
            drain(NCH - 1, 0)

        plsc.subcore_barrier()
        @pl.when(s < _NS - 1)
        def _():
            pltpu.sync_copy(acc.at[pl.ds(s * FULL, FULL)],
                            out_hbm.at[c, pl.ds(s * FULL, FULL)])
        @pl.when(s == _NS - 1)
        def _():
            pltpu.sync_copy(acc.at[pl.ds(s * FULL, LAST)],
                            out_hbm.at[c, pl.ds(s * FULL, LAST)])
        if with_deg:
            @pl.when(s < _NS - 1)
            def _():
                pltpu.sync_copy(dacc.at[pl.ds(s * FULL, FULL)], dzero)
                pltpu.sync_copy(
                    dzero, deg_hbm.at[pl.ds(c * N + s * FULL, FULL)])
            @pl.when(s == _NS - 1)
            def _():
                pltpu.sync_copy(dacc.at[pl.ds(s * FULL, LAST)],
                                dzero.at[pl.ds(0, LAST)])
                pltpu.sync_copy(
                    dzero.at[pl.ds(0, LAST)],
                    deg_hbm.at[pl.ds(c * N + s * FULL, LAST)])

    return agg


def _proj(x, Wl, Wr, bl):
    N, Din = x.shape
    Do = Wl.shape[1]
    BN = 1000

    def body(x_ref, wl_ref, wr_ref, b_ref, y_ref, z_ref):
        xb = x_ref[...]
        y_ref[...] = jnp.dot(xb, wl_ref[...],
                             preferred_element_type=jnp.float32)
        z_ref[...] = jnp.dot(xb, wr_ref[...],
                             preferred_element_type=jnp.float32) + b_ref[...]

    y, z = pl.pallas_call(
        body,
        grid=(N // BN,),
        in_specs=[
            pl.BlockSpec((BN, Din), lambda i: (i, 0)),
            pl.BlockSpec((Din, Do), lambda i: (0, 0)),
            pl.BlockSpec((Din, Do), lambda i: (0, 0)),
            pl.BlockSpec((1, Do), lambda i: (0, 0)),
        ],
        out_specs=[
            pl.BlockSpec((BN, Do), lambda i: (i, 0)),
            pl.BlockSpec((BN, Do), lambda i: (i, 0)),
        ],
        out_shape=[jax.ShapeDtypeStruct((N, Do), jnp.float32)] * 2,
    )(x, Wl, Wr, bl.reshape(1, -1))
    return y, z


def _comb_proj(p0, p1, z, d0, d1, Wl, Wr, bl, emit_h):
    N, Dh = z.shape
    Do = Wr.shape[1]
    BN = 1000

    def body(p0_ref, p1_ref, z_ref, d0_ref, d1_ref, wl_ref, wr_ref, b_ref,
             y_ref, z2_ref):
        deg = jnp.maximum(d0_ref[...] + d1_ref[...], 1.0)
        h = jnp.maximum((p0_ref[...] + p1_ref[...]) / deg + z_ref[...], 0.0)
        if emit_h:
            y_ref[...] = h
        else:
            y_ref[...] = jnp.dot(h, wl_ref[...],
                                 preferred_element_type=jnp.float32)
        z2_ref[...] = jnp.dot(h, wr_ref[...],
                              preferred_element_type=jnp.float32) + b_ref[...]

    return pl.pallas_call(
        body,
        grid=(N // BN,),
        in_specs=[
            pl.BlockSpec((BN, Dh), lambda i: (i, 0)),
            pl.BlockSpec((BN, Dh), lambda i: (i, 0)),
            pl.BlockSpec((BN, Dh), lambda i: (i, 0)),
            pl.BlockSpec((BN, 1), lambda i: (i, 0)),
            pl.BlockSpec((BN, 1), lambda i: (i, 0)),
            pl.BlockSpec(Wl.shape, lambda i: (0, 0)),
            pl.BlockSpec((Dh, Do), lambda i: (0, 0)),
            pl.BlockSpec((1, Do), lambda i: (0, 0)),
        ],
        out_specs=[
            pl.BlockSpec((BN, Dh), lambda i: (i, 0)),
            pl.BlockSpec((BN, Do), lambda i: (i, 0)),
        ],
        out_shape=[jax.ShapeDtypeStruct((N, Dh), jnp.float32),
                   jax.ShapeDtypeStruct((N, Do), jnp.float32)],
    )(p0, p1, z, d0, d1, Wl, Wr, bl.reshape(1, -1))


def _final(p0, p1, z2, d0, d1, Wl):
    N, Dh = p0.shape
    Do = Wl.shape[1]
    BN = 1000

    def body(p0_ref, p1_ref, z2_ref, d0_ref, d1_ref, wl_ref, o_ref):
        deg = jnp.maximum(d0_ref[...] + d1_ref[...], 1.0)
        m = (p0_ref[...] + p1_ref[...]) / deg
        u = (jnp.dot(m, wl_ref[...], preferred_element_type=jnp.float32)
             + z2_ref[...])
        mx = jnp.max(u, axis=1, keepdims=True)
        e = u - mx
        o_ref[...] = e - jnp.log(jnp.sum(jnp.exp(e), axis=1, keepdims=True))

    return pl.pallas_call(
        body,
        grid=(N // BN,),
        in_specs=[
            pl.BlockSpec((BN, Dh), lambda i: (i, 0)),
            pl.BlockSpec((BN, Dh), lambda i: (i, 0)),
            pl.BlockSpec((BN, Do), lambda i: (i, 0)),
            pl.BlockSpec((BN, 1), lambda i: (i, 0)),
            pl.BlockSpec((BN, 1), lambda i: (i, 0)),
            pl.BlockSpec((Dh, Do), lambda i: (0, 0)),
        ],
        out_specs=pl.BlockSpec((BN, Do), lambda i: (i, 0)),
        out_shape=jax.ShapeDtypeStruct((N, Do), jnp.float32),
    )(p0, p1, z2, d0, d1, Wl)


def kernel(x, edge_index, Wl0, bl0, Wr0, Wl1, bl1, Wr1, Wl2, bl2, Wr2):
    N, Din = x.shape
    E = edge_index.shape[1]
    Dh = Wl0.shape[1]
    NP = ((N + 128 * _NS - 1) // (128 * _NS)) * (128 * _NS)
    NW = _NC * _NS
    C = 80
    src = edge_index[0]
    dst = edge_index[1].reshape(NW, E // (NW * C), C)

    agg_deg = _make_agg(N, Dh, E, True, NP)
    agg_h = _make_agg(N, Dh, E, False, NP)

    y, z = _proj(x, Wl0, Wr0, bl0)
    p, degf = agg_deg(y, src, dst)
    degp = degf.reshape(_NC, N)
    d0 = degp[0].reshape(N, 1)
    d1 = degp[1].reshape(N, 1)

    y, z = _comb_proj(p[0], p[1], z, d0, d1, Wl1, Wr1, bl1, emit_h=False)
    (p,) = agg_h(y, src, dst)

    h2, z2 = _comb_proj(p[0], p[1], z, d0, d1, Wl2, Wr2, bl2, emit_h=True)
    (p,) = agg_h(h2, src, dst)
    return _final(p[0], p[1], z2, d0, d1, Wl2)
